# baseline (speedup 1.0000x reference)
import functools
import math

import jax
import jax.numpy as jnp
from jax import lax
from jax.experimental import pallas as pl
from jax.experimental.pallas import tpu as pltpu
from jax.experimental.pallas import tpu_sc as plsc

_SQRT_2_OVER_PI = math.sqrt(2.0 / math.pi)
_A = 0.044715
_B1 = -2.0 * _SQRT_2_OVER_PI
_B3 = -2.0 * _SQRT_2_OVER_PI * _A

_NC = 2
_NS = 16
_NW = _NC * _NS
_LANES = 16

_ROWS = 4096
_COLS = 2048
_SC_ROWS = 1024
_TC_ROWS = _ROWS - _SC_ROWS
_TC_BLOCK_ROWS = 1024

_SC_N = _SC_ROWS * _COLS
_TC_ELEMS = _TC_ROWS * _COLS
_CHUNK = 16384
_UNROLL = 8


def _tc_gelu_block(x_ref, o_ref):
    x = x_ref[...]
    inner = _SQRT_2_OVER_PI * (x + 0.044715 * (x * x * x))
    o_ref[...] = 0.5 * x * (1.0 + jnp.tanh(inner))


def _tc_gelu_partial(x2):
    grid = (_TC_ROWS // _TC_BLOCK_ROWS,)
    return pl.pallas_call(
        _tc_gelu_block,
        grid=grid,
        in_specs=[pl.BlockSpec((_TC_BLOCK_ROWS, _COLS), lambda i: (i, 0))],
        out_specs=pl.BlockSpec((_TC_BLOCK_ROWS, _COLS), lambda i: (i, 0)),
        out_shape=jax.ShapeDtypeStruct((_ROWS, _COLS), x2.dtype),
    )(x2)


def _gelu_slices(in_v, out_v, s0):
    for k in range(_UNROLL):
        sl = pl.ds(s0 + k * _LANES, _LANES)
        x = in_v[sl]
        x2 = x * x
        u = x * (_B1 + _B3 * x2)
        out_v[sl] = x / (1.0 + jnp.exp(u))


def _sc_gelu_body(x_hbm, o_hbm, in0, in1, out0, out1, si0, si1, so0, so1):
    wid = lax.axis_index("s") * _NC + lax.axis_index("c")
    per_w = _SC_N // _NW
    base_in = _TC_ELEMS + wid * per_w
    base_out = wid * per_w
    n_chunks = per_w // _CHUNK
    n_slices = _CHUNK // (_LANES * _UNROLL)

    bufs = ((in0, si0, out0, so0), (in1, si1, out1, so1))

    pltpu.async_copy(x_hbm.at[pl.ds(base_in, _CHUNK)], in0, si0)
    pltpu.async_copy(x_hbm.at[pl.ds(base_in + _CHUNK, _CHUNK)], in1, si1)

    def pair(i, carry):
        c0 = 2 * i
        for b in range(2):
            ibuf, isem, obuf, osem = bufs[b]
            c = c0 + b
            pltpu.make_async_copy(x_hbm.at[pl.ds(0, _CHUNK)], ibuf, isem).wait()
            @pl.when(i > 0)
            def _drain():
                pltpu.make_async_copy(
                    obuf, o_hbm.at[pl.ds(0, _CHUNK)], osem).wait()

            def inner(si, c2):
                _gelu_slices(ibuf, obuf, si * (_LANES * _UNROLL))
                return c2

            lax.fori_loop(0, n_slices, inner, 0, unroll=1)
            pltpu.async_copy(
                obuf, o_hbm.at[pl.ds(base_out + c * _CHUNK, _CHUNK)], osem)

            @pl.when(c + 2 < n_chunks)
            def _prefetch():
                pltpu.async_copy(
                    x_hbm.at[pl.ds(base_in + (c + 2) * _CHUNK, _CHUNK)],
                    ibuf, isem)
        return carry

    lax.fori_loop(0, n_chunks // 2, pair, 0, unroll=1)
    pltpu.make_async_copy(out0, o_hbm.at[pl.ds(0, _CHUNK)], so0).wait()
    pltpu.make_async_copy(out1, o_hbm.at[pl.ds(0, _CHUNK)], so1).wait()


def _sc_gelu(x_flat):
    mesh = plsc.VectorSubcoreMesh(core_axis_name="c", subcore_axis_name="s")
    f = functools.partial(
        pl.kernel,
        mesh=mesh,
        out_type=jax.ShapeDtypeStruct((_SC_N,), jnp.float32),
        scratch_types=[
            pltpu.VMEM((_CHUNK,), jnp.float32),
            pltpu.VMEM((_CHUNK,), jnp.float32),
            pltpu.VMEM((_CHUNK,), jnp.float32),
            pltpu.VMEM((_CHUNK,), jnp.float32),
            pltpu.SemaphoreType.DMA,
            pltpu.SemaphoreType.DMA,
            pltpu.SemaphoreType.DMA,
            pltpu.SemaphoreType.DMA,
        ],
    )(_sc_gelu_body)
    return f(x_flat)


def kernel(x, log_k_ramp, log_g_high):
    del log_k_ramp, log_g_high
    x2 = x.reshape(_ROWS, _COLS)
    y_sc = _sc_gelu(x.reshape(-1))
    y_tc_full = _tc_gelu_partial(x2)
    y = lax.dynamic_update_slice(
        y_tc_full, y_sc.reshape(_SC_ROWS, _COLS), (_TC_ROWS, 0))
    return y.reshape(x.shape)


# speedup vs baseline: 1.2256x; 1.2256x over previous
import functools
import math

import jax
import jax.numpy as jnp
from jax import lax
from jax.experimental import pallas as pl
from jax.experimental.pallas import tpu as pltpu
from jax.experimental.pallas import tpu_sc as plsc

_SQRT_2_OVER_PI = math.sqrt(2.0 / math.pi)
_A = 0.044715
_B1 = -2.0 * _SQRT_2_OVER_PI
_B3 = -2.0 * _SQRT_2_OVER_PI * _A

_NC = 2
_NS = 16
_NW = _NC * _NS
_LANES = 16

_ROWS = 4096
_COLS = 2048
_SC_ROWS = 1024
_TC_ROWS = _ROWS - _SC_ROWS
_TC_BLOCK_ROWS = 1024

_STRIPE = 8
_N_STRIPES = _SC_ROWS // _STRIPE
_STRIPES_PER_W = _N_STRIPES // _NW
_UNROLL = 8


def _tc_gelu_block(x_ref, o_ref):
    x = x_ref[...]
    inner = _SQRT_2_OVER_PI * (x + 0.044715 * (x * x * x))
    o_ref[...] = 0.5 * x * (1.0 + jnp.tanh(inner))


def _tc_gelu_partial(x2):
    grid = (_TC_ROWS // _TC_BLOCK_ROWS,)
    return pl.pallas_call(
        _tc_gelu_block,
        grid=grid,
        in_specs=[pl.BlockSpec((_TC_BLOCK_ROWS, _COLS), lambda i: (i, 0))],
        out_specs=pl.BlockSpec((_TC_BLOCK_ROWS, _COLS), lambda i: (i, 0)),
        out_shape=jax.ShapeDtypeStruct((_ROWS, _COLS), x2.dtype),
    )(x2)


def _gelu_row(in_v, out_v, r, c0):
    for k in range(_UNROLL):
        sl = pl.ds(c0 + k * _LANES, _LANES)
        x = in_v[r, sl]
        x2 = x * x
        u = x * (_B1 + _B3 * x2)
        out_v[r, sl] = x / (1.0 + jnp.exp(u))


def _sc_gelu_body(x_hbm, o_hbm, in0, in1, out0, out1, si0, si1, so0, so1):
    wid = lax.axis_index("s") * _NC + lax.axis_index("c")
    s_base = wid * _STRIPES_PER_W

    bufs = ((in0, si0, out0, so0), (in1, si1, out1, so1))

    def in_row(c):
        return _TC_ROWS + (s_base + c) * _STRIPE

    def out_row(c):
        return (s_base + c) * _STRIPE

    pltpu.async_copy(x_hbm.at[pl.ds(in_row(0), _STRIPE), :], in0, si0)
    pltpu.async_copy(x_hbm.at[pl.ds(in_row(1), _STRIPE), :], in1, si1)

    def pair(i, carry):
        c0 = 2 * i
        for b in range(2):
            ibuf, isem, obuf, osem = bufs[b]
            c = c0 + b
            pltpu.make_async_copy(
                x_hbm.at[pl.ds(0, _STRIPE), :], ibuf, isem).wait()
            @pl.when(i > 0)
            def _drain():
                pltpu.make_async_copy(
                    obuf, o_hbm.at[pl.ds(0, _STRIPE), :], osem).wait()

            def inner(si, c2):
                for r in range(_STRIPE):
                    _gelu_row(ibuf, obuf, r, si * (_LANES * _UNROLL))
                return c2

            lax.fori_loop(0, _COLS // (_LANES * _UNROLL), inner, 0, unroll=1)
            pltpu.async_copy(
                obuf, o_hbm.at[pl.ds(out_row(c), _STRIPE), :], osem)

            @pl.when(c + 2 < _STRIPES_PER_W)
            def _prefetch():
                pltpu.async_copy(
                    x_hbm.at[pl.ds(in_row(c + 2), _STRIPE), :], ibuf, isem)
        return carry

    lax.fori_loop(0, _STRIPES_PER_W // 2, pair, 0, unroll=1)
    pltpu.make_async_copy(out0, o_hbm.at[pl.ds(0, _STRIPE), :], so0).wait()
    pltpu.make_async_copy(out1, o_hbm.at[pl.ds(0, _STRIPE), :], so1).wait()


def _sc_gelu(x2):
    mesh = plsc.VectorSubcoreMesh(core_axis_name="c", subcore_axis_name="s")
    f = functools.partial(
        pl.kernel,
        mesh=mesh,
        out_type=jax.ShapeDtypeStruct((_SC_ROWS, _COLS), jnp.float32),
        scratch_types=[
            pltpu.VMEM((_STRIPE, _COLS), jnp.float32),
            pltpu.VMEM((_STRIPE, _COLS), jnp.float32),
            pltpu.VMEM((_STRIPE, _COLS), jnp.float32),
            pltpu.VMEM((_STRIPE, _COLS), jnp.float32),
            pltpu.SemaphoreType.DMA,
            pltpu.SemaphoreType.DMA,
            pltpu.SemaphoreType.DMA,
            pltpu.SemaphoreType.DMA,
        ],
        compiler_params=pltpu.CompilerParams(use_tc_tiling_on_sc=True),
    )(_sc_gelu_body)
    return f(x2)


def kernel(x, log_k_ramp, log_g_high):
    del log_k_ramp, log_g_high
    x2 = x.reshape(_ROWS, _COLS)
    y_sc = _sc_gelu(x2)
    y_tc_full = _tc_gelu_partial(x2)
    y = lax.dynamic_update_slice(y_tc_full, y_sc, (_TC_ROWS, 0))
    return y.reshape(x.shape)


# speedup vs baseline: 3.3638x; 2.7446x over previous
import math

import jax
import jax.numpy as jnp
from jax.experimental import pallas as pl

_SQRT_2_OVER_PI = math.sqrt(2.0 / math.pi)

_BLOCK_ROWS = 1024


def _gelu_block_kernel(x_ref, o_ref):
    x = x_ref[...]
    inner = _SQRT_2_OVER_PI * (x + 0.044715 * (x * x * x))
    o_ref[...] = 0.5 * x * (1.0 + jnp.tanh(inner))


def kernel(x, log_k_ramp, log_g_high):
    del log_k_ramp, log_g_high
    orig_shape = x.shape
    x2 = x.reshape(-1, orig_shape[-1])
    rows, cols = x2.shape
    grid = (rows // _BLOCK_ROWS,)
    y2 = pl.pallas_call(
        _gelu_block_kernel,
        grid=grid,
        in_specs=[pl.BlockSpec((_BLOCK_ROWS, cols), lambda i: (i, 0))],
        out_specs=pl.BlockSpec((_BLOCK_ROWS, cols), lambda i: (i, 0)),
        out_shape=jax.ShapeDtypeStruct((rows, cols), x.dtype),
    )(x2)
    return y2.reshape(orig_shape)
